# baseline (speedup 1.0000x reference)
import functools

import jax
import jax.numpy as jnp
from jax import lax
from jax.experimental import pallas as pl
from jax.experimental.pallas import tpu as pltpu
from jax.experimental.pallas import tpu_sc as plsc

_NC = 2
_NS = 16
_NW = _NC * _NS
_LANES = 16
_NBUF = 4
_CH = 64


@functools.lru_cache(maxsize=None)
def _build(n_rows, d, s):
    ch = _CH
    n_per_w = n_rows // _NW
    n_chunks = n_per_w // ch
    n_super = n_chunks // _NBUF
    assert n_per_w % ch == 0 and n_chunks % _NBUF == 0 and n_super >= 2
    assert ch <= s
    mesh = plsc.VectorSubcoreMesh(
        core_axis_name="c", subcore_axis_name="s",
        num_cores=_NC, num_subcores=_NS,
    )

    scratch = (
        tuple(pltpu.VMEM((ch,), jnp.int32) for _ in range(_NBUF)),
        tuple(pltpu.VMEM((ch, 128), jnp.float32) for _ in range(_NBUF)),
        tuple(pltpu.VMEM((ch, d), jnp.float32) for _ in range(_NBUF)),
        pltpu.VMEM((s, d), jnp.float32),
        tuple(pltpu.SemaphoreType.DMA for _ in range(_NBUF)),
        tuple(pltpu.SemaphoreType.DMA for _ in range(_NBUF)),
        tuple(pltpu.SemaphoreType.DMA for _ in range(_NBUF)),
    )

    @functools.partial(
        pl.kernel,
        out_type=jax.ShapeDtypeStruct((n_rows, d), jnp.float32),
        mesh=mesh,
        scratch_types=scratch,
    )
    def emb_kernel(tok_hbm, table_hbm, pos_hbm, out_hbm,
                   idx_v, rows_v, out_v, pos_v, gsem, ssem, isem):
        wid = lax.axis_index("s") * _NC + lax.axis_index("c")
        base = wid * n_per_w
        pltpu.sync_copy(pos_hbm, pos_v)

        def start_idx(g, b):
            pltpu.async_copy(tok_hbm.at[pl.ds(base + g * ch, ch)], idx_v[b], isem[b])

        def wait_idx(b):
            pltpu.make_async_copy(tok_hbm.at[pl.ds(0, ch)], idx_v[b], isem[b]).wait()

        def start_gather(b):
            pltpu.async_copy(table_hbm.at[idx_v[b]], rows_v[b], gsem[b])

        def wait_gather(b):
            pltpu.make_async_copy(table_hbm.at[idx_v[b]], rows_v[b], gsem[b]).wait()

        def start_store(g, b):
            pltpu.async_copy(out_v[b], out_hbm.at[pl.ds(base + g * ch, ch)], ssem[b])

        def wait_store(b):
            pltpu.make_async_copy(out_v[b], out_hbm.at[pl.ds(0, ch)], ssem[b]).wait()

        def compute(g, b):
            phase = (g * ch) % s
            rows_b, out_b = rows_v[b], out_v[b]

            def row_body(r, carry):
                p0 = phase + r
                p = jnp.where(p0 >= s, p0 - s, p0)
                for c in range(d // _LANES):
                    sl = pl.ds(c * _LANES, _LANES)
                    out_b[r, sl] = rows_b[r, sl] + pos_v[p, sl]
                return carry

            lax.fori_loop(0, ch, row_body, 0)

        def step(g, b, *, idx_next=True, store_wait=True, gather_next=True):
            wait_gather(b)
            if idx_next:
                start_idx(g + _NBUF, b)
            if store_wait:
                wait_store((b + _NBUF - 1) % _NBUF)
            if gather_next:
                hb = (b + _NBUF - 1) % _NBUF
                wait_idx(hb)
                start_gather(hb)
            compute(g, b)
            start_store(g, b)

        for b in range(_NBUF):
            start_idx(b, b)
        for b in range(_NBUF - 1):
            wait_idx(b)
            start_gather(b)
        step(0, 0, store_wait=False)
        for b in range(1, _NBUF):
            step(b, b)

        def super_body(go, carry):
            g0 = go * _NBUF
            for b in range(_NBUF):
                step(g0 + b, b)
            return carry

        lax.fori_loop(1, n_super - 1, super_body, 0)

        g0 = (n_super - 1) * _NBUF
        step(g0, 0, idx_next=False)
        for b in range(1, _NBUF):
            step(g0 + b, b, idx_next=False, gather_next=False)
        wait_store(_NBUF - 1)

    return emb_kernel


def kernel(tokens, token_embedding, position_embedding):
    b, s = tokens.shape
    _, d = token_embedding.shape
    flat = tokens.reshape(-1).astype(jnp.int32)
    table128 = jnp.pad(token_embedding, ((0, 0), (0, 128 - d)))
    fn = _build(b * s, d, s)
    out = fn(flat, table128, position_embedding[:s])
    return out.reshape(b, s, d)


# speedup vs baseline: 1.5916x; 1.5916x over previous
import functools

import jax
import jax.numpy as jnp
from jax import lax
from jax.experimental import pallas as pl
from jax.experimental.pallas import tpu as pltpu
from jax.experimental.pallas import tpu_sc as plsc

_NC = 2
_NS = 16
_NW = _NC * _NS
_LANES = 16
_NBUF = 2
_CH = 200


@functools.lru_cache(maxsize=None)
def _build(n_rows, d, s):
    ch = _CH
    n_per_w = n_rows // _NW
    n_chunks = n_per_w // ch
    n_super = n_chunks // _NBUF
    assert n_per_w % ch == 0 and n_chunks % _NBUF == 0 and n_super >= 2
    assert ch <= s or ch % s == 0
    mesh = plsc.VectorSubcoreMesh(
        core_axis_name="c", subcore_axis_name="s",
        num_cores=_NC, num_subcores=_NS,
    )

    scratch = (
        tuple(pltpu.VMEM((ch,), jnp.int32) for _ in range(_NBUF)),
        tuple(pltpu.VMEM((ch, 128), jnp.float32) for _ in range(_NBUF)),
        tuple(pltpu.VMEM((ch, d), jnp.float32) for _ in range(_NBUF)),
        pltpu.VMEM((s, d), jnp.float32),
        tuple(pltpu.SemaphoreType.DMA for _ in range(_NBUF)),
        tuple(pltpu.SemaphoreType.DMA for _ in range(_NBUF)),
        tuple(pltpu.SemaphoreType.DMA for _ in range(_NBUF)),
    )

    @functools.partial(
        pl.kernel,
        out_type=jax.ShapeDtypeStruct((n_rows, d), jnp.float32),
        mesh=mesh,
        scratch_types=scratch,
    )
    def emb_kernel(tok_hbm, table_hbm, pos_hbm, out_hbm,
                   idx_v, rows_v, out_v, pos_v, gsem, ssem, isem):
        wid = lax.axis_index("s") * _NC + lax.axis_index("c")
        base = wid * n_per_w
        pltpu.sync_copy(pos_hbm, pos_v)

        def start_idx(g, b):
            pltpu.async_copy(tok_hbm.at[pl.ds(base + g * ch, ch)], idx_v[b], isem[b])

        def wait_idx(b):
            pltpu.make_async_copy(tok_hbm.at[pl.ds(0, ch)], idx_v[b], isem[b]).wait()

        def start_gather(b):
            pltpu.async_copy(table_hbm.at[idx_v[b]], rows_v[b], gsem[b])

        def wait_gather(b):
            pltpu.make_async_copy(table_hbm.at[idx_v[b]], rows_v[b], gsem[b]).wait()

        def start_store(g, b):
            pltpu.async_copy(out_v[b], out_hbm.at[pl.ds(base + g * ch, ch)], ssem[b])

        def wait_store(b):
            pltpu.make_async_copy(out_v[b], out_hbm.at[pl.ds(0, ch)], ssem[b]).wait()

        def compute(g, b):
            rows_b, out_b = rows_v[b], out_v[b]

            def row_body(r, carry):
                if ch % s == 0:
                    p = r % s if ch > s else r
                else:
                    p0 = (g * ch) % s + r
                    p = jnp.where(p0 >= s, p0 - s, p0)
                for c in range(d // _LANES):
                    sl = pl.ds(c * _LANES, _LANES)
                    out_b[r, sl] = rows_b[r, sl] + pos_v[p, sl]
                return carry

            lax.fori_loop(0, ch, row_body, 0)

        def step(g, b, *, idx_next=True, store_wait=True, gather_next=True):
            wait_gather(b)
            if idx_next:
                start_idx(g + _NBUF, b)
            if store_wait:
                wait_store((b + _NBUF - 1) % _NBUF)
            if gather_next:
                hb = (b + _NBUF - 1) % _NBUF
                wait_idx(hb)
                start_gather(hb)
            compute(g, b)
            start_store(g, b)

        for b in range(_NBUF):
            start_idx(b, b)
        for b in range(_NBUF - 1):
            wait_idx(b)
            start_gather(b)
        step(0, 0, store_wait=False)
        for b in range(1, _NBUF):
            step(b, b)

        def super_body(go, carry):
            g0 = go * _NBUF
            for b in range(_NBUF):
                step(g0 + b, b)
            return carry

        lax.fori_loop(1, n_super - 1, super_body, 0)

        g0 = (n_super - 1) * _NBUF
        step(g0, 0, idx_next=False)
        for b in range(1, _NBUF):
            step(g0 + b, b, idx_next=False, gather_next=False)
        wait_store(_NBUF - 1)

    return emb_kernel


def kernel(tokens, token_embedding, position_embedding):
    b, s = tokens.shape
    _, d = token_embedding.shape
    flat = tokens.reshape(-1).astype(jnp.int32)
    table128 = jnp.pad(token_embedding, ((0, 0), (0, 128 - d)))
    fn = _build(b * s, d, s)
    out = fn(flat, table128, position_embedding[:s])
    return out.reshape(b, s, d)
